# baseline (speedup 1.0000x reference)
import jax
import jax.numpy as jnp
from jax import lax
from jax.experimental import pallas as pl
from jax.experimental.pallas import tpu as pltpu
from jax.experimental.pallas import tpu_sc as plsc

_DEPTH = 1000
_OUT = 64
_BATCH = 16384
_NC = 1
_NS = 16
_NW = _NC * _NS
_BPW = _BATCH // _NW
_CHUNK = 128
_NCH = _BPW // _CHUNK
_IDX_ROWS = _BATCH // _CHUNK
_VREG = 16
_NV = _OUT // _VREG


def _gather_body(t_hbm, b_hbm, idx_hbm, out_hbm,
                 idx_v, rows_v, bias_v, gsems, osems):
    wid = lax.axis_index("s") * _NC + lax.axis_index("c")
    pltpu.sync_copy(b_hbm, bias_v)
    pltpu.sync_copy(idx_hbm.at[pl.ds(wid * _NCH, _NCH)], idx_v)
    bias_regs = [bias_v[pl.ds(k * _VREG, _VREG)] for k in range(_NV)]

    def _fill_bias(i, _):
        for k in range(_NV):
            rows_v[i, pl.ds(k * _VREG, _VREG)] = bias_regs[k]
        return 0

    gathers = []
    for j in range(_NCH):
        lax.fori_loop(j * _CHUNK, (j + 1) * _CHUNK, _fill_bias, 0)
        gathers.append(
            pltpu.async_copy(t_hbm.at[idx_v.at[j]],
                             rows_v.at[pl.ds(j * _CHUNK, _CHUNK)], gsems.at[j],
                             add=True))
    writes = []
    for j in range(_NCH):
        gathers[j].wait()
        writes.append(
            pltpu.async_copy(rows_v.at[pl.ds(j * _CHUNK, _CHUNK)],
                             out_hbm.at[pl.ds(wid * _BPW + j * _CHUNK, _CHUNK)],
                             osems.at[j]))
    for cp in writes:
        cp.wait()


def kernel(input_batch, W, b):
    idx = jnp.reshape(input_batch.astype(jnp.int32), (_IDX_ROWS, _CHUNK))
    table = W.T

    mesh = plsc.VectorSubcoreMesh(core_axis_name="c", subcore_axis_name="s",
                                  num_cores=_NC, num_subcores=_NS)
    run = pl.kernel(
        _gather_body,
        out_type=jax.ShapeDtypeStruct((_BATCH, _OUT), jnp.float32),
        mesh=mesh,
        scratch_types=[
            pltpu.VMEM((_NCH, _CHUNK), jnp.int32),
            pltpu.VMEM((_BPW, _OUT), jnp.float32),
            pltpu.VMEM((_OUT,), jnp.float32),
            pltpu.SemaphoreType.DMA((_NCH,)),
            pltpu.SemaphoreType.DMA((_NCH,)),
        ],
        compiler_params=pltpu.CompilerParams(
            use_tc_tiling_on_sc=False,
            disable_bounds_checks=True,
            disable_semaphore_checks=True,
        ),
    )
    out = run(table, b, idx)
    return out[:, None, :]


# speedup vs baseline: 1.0063x; 1.0063x over previous
import jax
import jax.numpy as jnp
from jax import lax
from jax.experimental import pallas as pl
from jax.experimental.pallas import tpu as pltpu
from jax.experimental.pallas import tpu_sc as plsc

_DEPTH = 1000
_OUT = 64
_BATCH = 16384
_NC = 1
_NS = 16
_NW = _NC * _NS
_BPW = _BATCH // _NW
_CHUNK = 128
_NCH = _BPW // _CHUNK
_IDX_ROWS = _BATCH // _CHUNK
_VREG = 16
_NV = _OUT // _VREG


def _gather_body(t_hbm, b_hbm, idx_hbm, out_hbm,
                 idx_v, rows_v, bias_v, gsems, osems):
    wid = lax.axis_index("s") * _NC + lax.axis_index("c")
    pltpu.sync_copy(b_hbm, bias_v)
    pltpu.sync_copy(idx_hbm.at[pl.ds(wid * _NCH, _NCH)], idx_v)
    bias_regs = [bias_v[pl.ds(k * _VREG, _VREG)] for k in range(_NV)]

    def _fill_bias(i, _):
        for k in range(_NV):
            rows_v[i, pl.ds(k * _VREG, _VREG)] = bias_regs[k]
        return 0

    gathers = []
    for j in range(_NCH):
        lax.fori_loop(j * _CHUNK, (j + 1) * _CHUNK, _fill_bias, 0)
        gathers.append(
            pltpu.async_copy(t_hbm.at[idx_v.at[j]],
                             rows_v.at[pl.ds(j * _CHUNK, _CHUNK)], gsems.at[j],
                             add=True))
    writes = []
    for j in range(_NCH):
        gathers[j].wait()
        writes.append(
            pltpu.async_copy(rows_v.at[pl.ds(j * _CHUNK, _CHUNK)],
                             out_hbm.at[pl.ds(wid * _BPW + j * _CHUNK, _CHUNK)],
                             osems.at[j]))
    for cp in writes:
        cp.wait()


def kernel(input_batch, W, b):
    idx = jnp.reshape(input_batch.astype(jnp.int32), (_IDX_ROWS, _CHUNK))
    table = W.T

    mesh = plsc.VectorSubcoreMesh(core_axis_name="c", subcore_axis_name="s",
                                  num_cores=_NC, num_subcores=_NS)
    run = pl.kernel(
        _gather_body,
        out_type=jax.ShapeDtypeStruct((_BATCH, _OUT), jnp.float32),
        mesh=mesh,
        scratch_types=[
            pltpu.VMEM((_NCH, _CHUNK), jnp.int32),
            pltpu.VMEM((_BPW, _OUT), jnp.float32),
            pltpu.VMEM((_OUT,), jnp.float32),
            pltpu.SemaphoreType.DMA((_NCH,)),
            pltpu.SemaphoreType.DMA((_NCH,)),
        ],
        compiler_params=pltpu.CompilerParams(
            use_tc_tiling_on_sc=False,
            disable_bounds_checks=True,
            disable_semaphore_checks=True,
            skip_device_barrier=True,
        ),
    )
    out = run(table, b, idx)
    return out[:, None, :]


# speedup vs baseline: 1.0071x; 1.0008x over previous
import jax
import jax.numpy as jnp
from jax import lax
from jax.experimental import pallas as pl
from jax.experimental.pallas import tpu as pltpu
from jax.experimental.pallas import tpu_sc as plsc

_DEPTH = 1000
_OUT = 64
_BATCH = 16384
_NC = 1
_NS = 16
_NW = _NC * _NS
_BPW = _BATCH // _NW
_CHUNK = 128
_NCH = _BPW // _CHUNK
_IDX_ROWS = _BATCH // _CHUNK
_VREG = 16
_NV = _OUT // _VREG


def _gather_body(t_hbm, b_hbm, idx_hbm, out_hbm,
                 idx_v, rows_v, bias_v, gsems, osems):
    wid = lax.axis_index("s") * _NC + lax.axis_index("c")
    pltpu.sync_copy(b_hbm, bias_v)
    pltpu.sync_copy(idx_hbm.at[pl.ds(wid * _NCH, _NCH)], idx_v)
    bias_regs = [bias_v[pl.ds(k * _VREG, _VREG)] for k in range(_NV)]

    def _fill_bias(i, _):
        for k in range(_NV):
            rows_v[i, pl.ds(k * _VREG, _VREG)] = bias_regs[k]
        return 0

    gathers = []
    for j in range(_NCH):
        lax.fori_loop(j * _CHUNK, (j + 1) * _CHUNK, _fill_bias, 0)
        gathers.append(
            pltpu.async_copy(t_hbm.at[idx_v.at[j]],
                             rows_v.at[pl.ds(j * _CHUNK, _CHUNK)], gsems.at[j],
                             add=True))
    writes = []
    for j in range(_NCH):
        gathers[j].wait()
        writes.append(
            pltpu.async_copy(rows_v.at[pl.ds(j * _CHUNK, _CHUNK)],
                             out_hbm.at[pl.ds(wid * _BPW + j * _CHUNK, _CHUNK)],
                             osems.at[j]))
    for cp in writes:
        cp.wait()


def kernel(input_batch, W, b):
    idx = jnp.reshape(input_batch.astype(jnp.int32), (_IDX_ROWS, _CHUNK))
    table = W.T

    mesh = plsc.VectorSubcoreMesh(core_axis_name="c", subcore_axis_name="s",
                                  num_cores=_NC, num_subcores=_NS)
    run = pl.kernel(
        _gather_body,
        out_type=jax.ShapeDtypeStruct((_BATCH, _OUT), jnp.float32),
        mesh=mesh,
        scratch_types=[
            pltpu.VMEM((_NCH, _CHUNK), jnp.int32),
            pltpu.VMEM((_BPW, _OUT), jnp.float32),
            pltpu.VMEM((_OUT,), jnp.float32),
            pltpu.SemaphoreType.DMA((_NCH,)),
            pltpu.SemaphoreType.DMA((_NCH,)),
        ],
        compiler_params=pltpu.CompilerParams(
            use_tc_tiling_on_sc=False,
            disable_bounds_checks=True,
            disable_semaphore_checks=True,
        ),
    )
    out = run(table, b, idx)
    return out[:, None, :]
